# baseline (device time: 47350 ns/iter reference)
import jax
import jax.numpy as jnp
from jax import lax
from jax.experimental import pallas as pl
from jax.experimental.pallas import tpu as pltpu

N_DEV = 4
CAP = 204
LANES = 128


def kernel(x, router_W, route_idx, expert_W):
    n_tok, d = x.shape
    e_per, _, h = expert_W.shape

    def body(x_ref, rw_ref, idx_ref, ew_ref, out_ref,
             comm_ref, hists_ref,
             send_sems, recv_sems, hist_send_sems, hist_recv_sems):
        my_pos = lax.axis_index("i")
        right = lax.rem(my_pos + 1, N_DEV)

        barrier_sem = pltpu.get_barrier_semaphore()
        for dlt in range(1, N_DEV):
            tgt = lax.rem(my_pos + dlt, N_DEV)
            pl.semaphore_signal(barrier_sem, inc=1, device_id=(tgt,),
                                device_id_type=pl.DeviceIdType.MESH)
        pl.semaphore_wait(barrier_sem, N_DEV - 1)

        lane = lax.broadcasted_iota(jnp.int32, (n_tok, LANES), 1)
        onehot = (idx_ref[...] == lane).astype(jnp.float32)
        hist = jnp.sum(onehot, axis=0)
        hists_ref[pl.ds(my_pos, 1)] = hist[None]

        hist_sends = []
        for dlt in range(1, N_DEV):
            tgt = lax.rem(my_pos + dlt, N_DEV)
            snd = pltpu.make_async_remote_copy(
                src_ref=hists_ref.at[pl.ds(my_pos, 1)],
                dst_ref=hists_ref.at[pl.ds(my_pos, 1)],
                send_sem=hist_send_sems.at[dlt - 1],
                recv_sem=hist_recv_sems.at[dlt - 1],
                device_id=(tgt,),
                device_id_type=pl.DeviceIdType.MESH,
            )
            snd.start()
            hist_sends.append(snd)

        def hop_rdma(hop):
            return pltpu.make_async_remote_copy(
                src_ref=ew_ref if hop == 0 else comm_ref.at[hop - 1],
                dst_ref=comm_ref.at[hop],
                send_sem=send_sems.at[hop],
                recv_sem=recv_sems.at[hop],
                device_id=(right,),
                device_id_type=pl.DeviceIdType.MESH,
            )

        rdma0 = hop_rdma(0)
        rdma0.start()

        for dlt in range(1, N_DEV):
            org = lax.rem(my_pos + N_DEV - dlt, N_DEV)
            rcv = pltpu.make_async_remote_copy(
                src_ref=hists_ref.at[pl.ds(org, 1)],
                dst_ref=hists_ref.at[pl.ds(org, 1)],
                send_sem=hist_send_sems.at[dlt - 1],
                recv_sem=hist_recv_sems.at[dlt - 1],
                device_id=(org,),
                device_id_type=pl.DeviceIdType.MESH,
            )
            rcv.wait_recv()
        for snd in hist_sends:
            snd.wait_send()

        allh = hists_ref[...]
        rowq = lax.broadcasted_iota(jnp.int32, (N_DEV, LANES), 0)
        base = jnp.sum(jnp.where(rowq < my_pos, allh, 0.0), axis=0)
        rr = lax.broadcasted_iota(jnp.int32, (n_tok, n_tok), 0)
        cc = lax.broadcasted_iota(jnp.int32, (n_tok, n_tok), 1)
        tril = (cc < rr).astype(jnp.float32)
        cum = jnp.dot(tril, onehot, preferred_element_type=jnp.float32)
        pos = cum + base[None, :]
        pos_tok = jnp.sum(pos * onehot, axis=1)
        kept = (pos_tok < CAP).astype(jnp.float32)
        mask = onehot * kept[:, None]

        xv = x_ref[...]

        def chunk_out(wref, org):
            res = None
            for j in range(e_per):
                mcol = jnp.sum(
                    jnp.where(lane == org * e_per + j, mask, 0.0), axis=1)
                xm = xv * mcol[:, None]
                p = jnp.dot(xm, wref[j], preferred_element_type=jnp.float32)
                res = p if res is None else res + p
            return res

        acc = chunk_out(ew_ref, my_pos)
        rdma0.wait()
        for hop in range(1, N_DEV - 1):
            rd = hop_rdma(hop)
            rd.start()
            acc = acc + chunk_out(comm_ref.at[hop - 1],
                                  lax.rem(my_pos + N_DEV - hop, N_DEV))
            rd.wait()
        acc = acc + chunk_out(comm_ref.at[N_DEV - 2],
                              lax.rem(my_pos + 1, N_DEV))
        out_ref[...] = acc

    return pl.pallas_call(
        body,
        out_shape=jax.ShapeDtypeStruct((n_tok, h), jnp.float32),
        in_specs=[pl.BlockSpec(memory_space=pltpu.VMEM)] * 4,
        out_specs=pl.BlockSpec(memory_space=pltpu.VMEM),
        scratch_shapes=[
            pltpu.VMEM((N_DEV - 1, e_per, d, h), jnp.float32),
            pltpu.VMEM((N_DEV, LANES), jnp.float32),
            pltpu.SemaphoreType.DMA((N_DEV - 1,)),
            pltpu.SemaphoreType.DMA((N_DEV - 1,)),
            pltpu.SemaphoreType.DMA((N_DEV - 1,)),
            pltpu.SemaphoreType.DMA((N_DEV - 1,)),
        ],
        compiler_params=pltpu.CompilerParams(collective_id=0),
    )(x, router_W, route_idx, expert_W)


# device time: 33451 ns/iter; 1.4155x vs baseline; 1.4155x over previous
import jax
import jax.numpy as jnp
from jax import lax
from jax.experimental import pallas as pl
from jax.experimental.pallas import tpu as pltpu

N_DEV = 4
CAP = 204
LANES = 128


def kernel(x, router_W, route_idx, expert_W):
    n_tok, d = x.shape
    e_per, _, h = expert_W.shape

    def body(x_ref, rw_ref, idx_ref, ew_ref, out_ref,
             comm_ref, hists_ref,
             send_sems, recv_sems, hist_send_sems, hist_recv_sems):
        my_pos = lax.axis_index("i")

        barrier_sem = pltpu.get_barrier_semaphore()
        for dlt in range(1, N_DEV):
            tgt = lax.rem(my_pos + dlt, N_DEV)
            pl.semaphore_signal(barrier_sem, inc=1, device_id=(tgt,),
                                device_id_type=pl.DeviceIdType.MESH)
        pl.semaphore_wait(barrier_sem, N_DEV - 1)

        lane = lax.broadcasted_iota(jnp.int32, (n_tok, LANES), 1)
        onehot = (idx_ref[...] == lane).astype(jnp.float32)
        hist = jnp.sum(onehot, axis=0)
        hists_ref[pl.ds(my_pos, 1)] = hist[None]

        hist_sends = []
        for dlt in range(1, N_DEV):
            tgt = lax.rem(my_pos + dlt, N_DEV)
            snd = pltpu.make_async_remote_copy(
                src_ref=hists_ref.at[pl.ds(my_pos, 1)],
                dst_ref=hists_ref.at[pl.ds(my_pos, 1)],
                send_sem=hist_send_sems.at[dlt - 1],
                recv_sem=hist_recv_sems.at[dlt - 1],
                device_id=(tgt,),
                device_id_type=pl.DeviceIdType.MESH,
            )
            snd.start()
            hist_sends.append(snd)

        w_sends = []
        for dlt in range(1, N_DEV):
            tgt = lax.rem(my_pos + dlt, N_DEV)
            snd = pltpu.make_async_remote_copy(
                src_ref=ew_ref,
                dst_ref=comm_ref.at[dlt - 1],
                send_sem=send_sems.at[dlt - 1],
                recv_sem=recv_sems.at[dlt - 1],
                device_id=(tgt,),
                device_id_type=pl.DeviceIdType.MESH,
            )
            snd.start()
            w_sends.append(snd)

        for dlt in range(1, N_DEV):
            org = lax.rem(my_pos + N_DEV - dlt, N_DEV)
            rcv = pltpu.make_async_remote_copy(
                src_ref=hists_ref.at[pl.ds(org, 1)],
                dst_ref=hists_ref.at[pl.ds(org, 1)],
                send_sem=hist_send_sems.at[dlt - 1],
                recv_sem=hist_recv_sems.at[dlt - 1],
                device_id=(org,),
                device_id_type=pl.DeviceIdType.MESH,
            )
            rcv.wait_recv()
        for snd in hist_sends:
            snd.wait_send()

        allh = hists_ref[...]
        rowq = lax.broadcasted_iota(jnp.int32, (N_DEV, LANES), 0)
        base = jnp.sum(jnp.where(rowq < my_pos, allh, 0.0), axis=0)
        rr = lax.broadcasted_iota(jnp.int32, (n_tok, n_tok), 0)
        cc = lax.broadcasted_iota(jnp.int32, (n_tok, n_tok), 1)
        tril = (cc < rr).astype(jnp.float32)
        cum = jnp.dot(tril, onehot, preferred_element_type=jnp.float32)
        pos = cum + base[None, :]
        pos_tok = jnp.sum(pos * onehot, axis=1)
        kept = (pos_tok < CAP).astype(jnp.float32)
        mask = onehot * kept[:, None]

        xv = x_ref[...]

        def chunk_out(wref, org):
            res = None
            for j in range(e_per):
                mcol = jnp.sum(
                    jnp.where(lane == org * e_per + j, mask, 0.0), axis=1)
                xm = xv * mcol[:, None]
                p = jnp.dot(xm, wref[j], preferred_element_type=jnp.float32)
                res = p if res is None else res + p
            return res

        acc = chunk_out(ew_ref, my_pos)
        for s in (0, 2, 1):
            w_sends[s].wait_recv()
            acc = acc + chunk_out(comm_ref.at[s],
                                  lax.rem(my_pos + N_DEV - (s + 1), N_DEV))
        for snd in w_sends:
            snd.wait_send()
        out_ref[...] = acc

    return pl.pallas_call(
        body,
        out_shape=jax.ShapeDtypeStruct((n_tok, h), jnp.float32),
        in_specs=[pl.BlockSpec(memory_space=pltpu.VMEM)] * 4,
        out_specs=pl.BlockSpec(memory_space=pltpu.VMEM),
        scratch_shapes=[
            pltpu.VMEM((N_DEV - 1, e_per, d, h), jnp.float32),
            pltpu.VMEM((N_DEV, LANES), jnp.float32),
            pltpu.SemaphoreType.DMA((N_DEV - 1,)),
            pltpu.SemaphoreType.DMA((N_DEV - 1,)),
            pltpu.SemaphoreType.DMA((N_DEV - 1,)),
            pltpu.SemaphoreType.DMA((N_DEV - 1,)),
        ],
        compiler_params=pltpu.CompilerParams(collective_id=0),
    )(x, router_W, route_idx, expert_W)


# device time: 22205 ns/iter; 2.1324x vs baseline; 1.5065x over previous
import jax
import jax.numpy as jnp
from jax import lax
from jax.experimental import pallas as pl
from jax.experimental.pallas import tpu as pltpu

N_DEV = 4
CAP = 204
LANES = 128


def kernel(x, router_W, route_idx, expert_W):
    n_tok, d = x.shape
    e_per, _, h = expert_W.shape

    def body(x_ref, rw_ref, idx_ref, ew_ref, out_ref,
             comm_ref, ewbf_ref, hists_ref,
             send_sems, recv_sems, hist_send_sems, hist_recv_sems):
        my_pos = lax.axis_index("i")

        barrier_sem = pltpu.get_barrier_semaphore()
        for dlt in range(1, N_DEV):
            tgt = lax.rem(my_pos + dlt, N_DEV)
            pl.semaphore_signal(barrier_sem, inc=1, device_id=(tgt,),
                                device_id_type=pl.DeviceIdType.MESH)
        ewbf_ref[...] = ew_ref[...].astype(jnp.bfloat16)
        pl.semaphore_wait(barrier_sem, N_DEV - 1)

        lane = lax.broadcasted_iota(jnp.int32, (n_tok, LANES), 1)
        onehot = (idx_ref[...] == lane).astype(jnp.float32)
        hist = jnp.sum(onehot, axis=0)
        hists_ref[pl.ds(my_pos, 1)] = hist[None]

        hist_sends = []
        for dlt in range(1, N_DEV):
            tgt = lax.rem(my_pos + dlt, N_DEV)
            snd = pltpu.make_async_remote_copy(
                src_ref=hists_ref.at[pl.ds(my_pos, 1)],
                dst_ref=hists_ref.at[pl.ds(my_pos, 1)],
                send_sem=hist_send_sems.at[dlt - 1],
                recv_sem=hist_recv_sems.at[dlt - 1],
                device_id=(tgt,),
                device_id_type=pl.DeviceIdType.MESH,
            )
            snd.start()
            hist_sends.append(snd)

        w_sends = []
        for dlt in range(1, N_DEV):
            tgt = lax.rem(my_pos + dlt, N_DEV)
            snd = pltpu.make_async_remote_copy(
                src_ref=ewbf_ref,
                dst_ref=comm_ref.at[dlt - 1],
                send_sem=send_sems.at[dlt - 1],
                recv_sem=recv_sems.at[dlt - 1],
                device_id=(tgt,),
                device_id_type=pl.DeviceIdType.MESH,
            )
            snd.start()
            w_sends.append(snd)

        for dlt in range(1, N_DEV):
            org = lax.rem(my_pos + N_DEV - dlt, N_DEV)
            rcv = pltpu.make_async_remote_copy(
                src_ref=hists_ref.at[pl.ds(org, 1)],
                dst_ref=hists_ref.at[pl.ds(org, 1)],
                send_sem=hist_send_sems.at[dlt - 1],
                recv_sem=hist_recv_sems.at[dlt - 1],
                device_id=(org,),
                device_id_type=pl.DeviceIdType.MESH,
            )
            rcv.wait_recv()
        for snd in hist_sends:
            snd.wait_send()

        allh = hists_ref[...]
        rowq = lax.broadcasted_iota(jnp.int32, (N_DEV, LANES), 0)
        base = jnp.sum(jnp.where(rowq < my_pos, allh, 0.0), axis=0)
        rr = lax.broadcasted_iota(jnp.int32, (n_tok, n_tok), 0)
        cc = lax.broadcasted_iota(jnp.int32, (n_tok, n_tok), 1)
        tril = (cc < rr).astype(jnp.float32)
        cum = jnp.dot(tril, onehot, preferred_element_type=jnp.float32)
        pos = cum + base[None, :]
        pos_tok = jnp.sum(pos * onehot, axis=1)
        kept = (pos_tok < CAP).astype(jnp.float32)
        mask = onehot * kept[:, None]

        xbf = x_ref[...].astype(jnp.bfloat16)

        def chunk_out(wref, org):
            res = None
            for j in range(e_per):
                mcol = jnp.sum(
                    jnp.where(lane == org * e_per + j, mask, 0.0), axis=1)
                xm = xbf * mcol.astype(jnp.bfloat16)[:, None]
                p = jnp.dot(xm, wref[j], preferred_element_type=jnp.float32)
                res = p if res is None else res + p
            return res

        acc = chunk_out(ewbf_ref, my_pos)
        for s in (0, 2, 1):
            w_sends[s].wait_recv()
            acc = acc + chunk_out(comm_ref.at[s],
                                  lax.rem(my_pos + N_DEV - (s + 1), N_DEV))
        for snd in w_sends:
            snd.wait_send()
        out_ref[...] = acc

    return pl.pallas_call(
        body,
        out_shape=jax.ShapeDtypeStruct((n_tok, h), jnp.float32),
        in_specs=[pl.BlockSpec(memory_space=pltpu.VMEM)] * 4,
        out_specs=pl.BlockSpec(memory_space=pltpu.VMEM),
        scratch_shapes=[
            pltpu.VMEM((N_DEV - 1, e_per, d, h), jnp.bfloat16),
            pltpu.VMEM((e_per, d, h), jnp.bfloat16),
            pltpu.VMEM((N_DEV, LANES), jnp.float32),
            pltpu.SemaphoreType.DMA((N_DEV - 1,)),
            pltpu.SemaphoreType.DMA((N_DEV - 1,)),
            pltpu.SemaphoreType.DMA((N_DEV - 1,)),
            pltpu.SemaphoreType.DMA((N_DEV - 1,)),
        ],
        compiler_params=pltpu.CompilerParams(collective_id=0),
    )(x, router_W, route_idx, expert_W)


# device time: 16534 ns/iter; 2.8638x vs baseline; 1.3430x over previous
import jax
import jax.numpy as jnp
from jax import lax
from jax.experimental import pallas as pl
from jax.experimental.pallas import tpu as pltpu

N_DEV = 4
CAP = 204
LANES = 128


def kernel(x, router_W, route_idx, expert_W):
    n_tok, d = x.shape
    e_per, _, h = expert_W.shape
    scale_base = N_DEV * e_per

    def body(x_ref, rw_ref, idx_ref, ew_ref, out_ref,
             comm_ref, q8_ref, hists_ref,
             send_sems, recv_sems, hist_send_sems, hist_recv_sems):
        my_pos = lax.axis_index("i")

        barrier_sem = pltpu.get_barrier_semaphore()
        for dlt in range(1, N_DEV):
            tgt = lax.rem(my_pos + dlt, N_DEV)
            pl.semaphore_signal(barrier_sem, inc=1, device_id=(tgt,),
                                device_id_type=pl.DeviceIdType.MESH)

        own_scales = []
        for j in range(e_per):
            w = ew_ref[j]
            s = (jnp.max(jnp.abs(w)) * (1.0 / 127.0)).astype(
                jnp.bfloat16).astype(jnp.float32)
            q8_ref[j] = jnp.clip(
                jnp.round(w * (1.0 / s)), -127.0, 127.0).astype(jnp.int8)
            own_scales.append(s)

        lane = lax.broadcasted_iota(jnp.int32, (n_tok, LANES), 1)
        onehot = (idx_ref[...] == lane).astype(jnp.float32)
        hist = jnp.sum(onehot, axis=0)
        l1 = lax.broadcasted_iota(jnp.int32, (1, LANES), 1)
        row = hist[None]
        for j, s in enumerate(own_scales):
            row = row + jnp.where(l1 == scale_base + j, s, 0.0)
        hists_ref[pl.ds(my_pos, 1)] = row

        pl.semaphore_wait(barrier_sem, N_DEV - 1)

        hist_sends = []
        for dlt in range(1, N_DEV):
            tgt = lax.rem(my_pos + dlt, N_DEV)
            snd = pltpu.make_async_remote_copy(
                src_ref=hists_ref.at[pl.ds(my_pos, 1)],
                dst_ref=hists_ref.at[pl.ds(my_pos, 1)],
                send_sem=hist_send_sems.at[dlt - 1],
                recv_sem=hist_recv_sems.at[dlt - 1],
                device_id=(tgt,),
                device_id_type=pl.DeviceIdType.MESH,
            )
            snd.start()
            hist_sends.append(snd)

        w_sends = []
        for dlt in range(1, N_DEV):
            tgt = lax.rem(my_pos + dlt, N_DEV)
            snd = pltpu.make_async_remote_copy(
                src_ref=q8_ref,
                dst_ref=comm_ref.at[dlt - 1],
                send_sem=send_sems.at[dlt - 1],
                recv_sem=recv_sems.at[dlt - 1],
                device_id=(tgt,),
                device_id_type=pl.DeviceIdType.MESH,
            )
            snd.start()
            w_sends.append(snd)

        for dlt in range(1, N_DEV):
            org = lax.rem(my_pos + N_DEV - dlt, N_DEV)
            rcv = pltpu.make_async_remote_copy(
                src_ref=hists_ref.at[pl.ds(org, 1)],
                dst_ref=hists_ref.at[pl.ds(org, 1)],
                send_sem=hist_send_sems.at[dlt - 1],
                recv_sem=hist_recv_sems.at[dlt - 1],
                device_id=(org,),
                device_id_type=pl.DeviceIdType.MESH,
            )
            rcv.wait_recv()
        for snd in hist_sends:
            snd.wait_send()

        allh = hists_ref[...]
        rowq = lax.broadcasted_iota(jnp.int32, (N_DEV, LANES), 0)
        base = jnp.sum(jnp.where(rowq < my_pos, allh, 0.0), axis=0)
        rr = lax.broadcasted_iota(jnp.int32, (n_tok, n_tok), 0)
        cc = lax.broadcasted_iota(jnp.int32, (n_tok, n_tok), 1)
        tril = (cc < rr).astype(jnp.float32)
        cum = jnp.dot(tril, onehot, preferred_element_type=jnp.float32)
        pos = cum + base[None, :]
        pos_tok = jnp.sum(pos * onehot, axis=1)
        kept = (pos_tok < CAP).astype(jnp.float32)
        mask = onehot * kept[:, None]

        xbf = x_ref[...].astype(jnp.bfloat16)

        def chunk_out(qref, org, scales):
            res = None
            for j in range(e_per):
                mcol = jnp.sum(
                    jnp.where(lane == org * e_per + j, mask, 0.0), axis=1)
                xm = xbf * (mcol * scales[j]).astype(jnp.bfloat16)[:, None]
                p = jnp.dot(xm, qref[j].astype(jnp.bfloat16),
                            preferred_element_type=jnp.float32)
                res = p if res is None else res + p
            return res

        acc = chunk_out(q8_ref, my_pos, own_scales)
        for s in (0, 2, 1):
            w_sends[s].wait_recv()
            org = lax.rem(my_pos + N_DEV - (s + 1), N_DEV)
            orow = jnp.sum(jnp.where(rowq == org, allh, 0.0), axis=0)
            acc = acc + chunk_out(
                comm_ref.at[s], org,
                [orow[scale_base + j] for j in range(e_per)])
        for snd in w_sends:
            snd.wait_send()
        out_ref[...] = acc

    return pl.pallas_call(
        body,
        out_shape=jax.ShapeDtypeStruct((n_tok, h), jnp.float32),
        in_specs=[pl.BlockSpec(memory_space=pltpu.VMEM)] * 4,
        out_specs=pl.BlockSpec(memory_space=pltpu.VMEM),
        scratch_shapes=[
            pltpu.VMEM((N_DEV - 1, e_per, d, h), jnp.int8),
            pltpu.VMEM((e_per, d, h), jnp.int8),
            pltpu.VMEM((N_DEV, LANES), jnp.float32),
            pltpu.SemaphoreType.DMA((N_DEV - 1,)),
            pltpu.SemaphoreType.DMA((N_DEV - 1,)),
            pltpu.SemaphoreType.DMA((N_DEV - 1,)),
            pltpu.SemaphoreType.DMA((N_DEV - 1,)),
        ],
        compiler_params=pltpu.CompilerParams(collective_id=0),
    )(x, router_W, route_idx, expert_W)


# device time: 16336 ns/iter; 2.8985x vs baseline; 1.0121x over previous
import jax
import jax.numpy as jnp
from jax import lax
from jax.experimental import pallas as pl
from jax.experimental.pallas import tpu as pltpu

N_DEV = 4
CAP = 204
LANES = 128


def kernel(x, router_W, route_idx, expert_W):
    n_tok, d = x.shape
    e_per, _, h = expert_W.shape
    scale_base = N_DEV * e_per

    def body(x_ref, rw_ref, idx_ref, ew_ref, out_ref,
             comm_ref, q8_ref, hists_ref,
             send_sems, recv_sems, hist_send_sems, hist_recv_sems):
        my_pos = lax.axis_index("i")

        barrier_sem = pltpu.get_barrier_semaphore()
        for dlt in range(1, N_DEV):
            tgt = lax.rem(my_pos + dlt, N_DEV)
            pl.semaphore_signal(barrier_sem, inc=1, device_id=(tgt,),
                                device_id_type=pl.DeviceIdType.MESH)

        own_scales = []
        for j in range(e_per):
            w = ew_ref[j]
            s = (jnp.max(jnp.abs(w)) * (1.0 / 127.0)).astype(
                jnp.bfloat16).astype(jnp.float32)
            q8_ref[j] = jnp.clip(
                jnp.round(w * (1.0 / s)), -127.0, 127.0).astype(jnp.int8)
            own_scales.append(s)

        lane = lax.broadcasted_iota(jnp.int32, (n_tok, LANES), 1)
        onehot = (idx_ref[...] == lane).astype(jnp.float32)
        hist = jnp.sum(onehot, axis=0)
        l1 = lax.broadcasted_iota(jnp.int32, (1, LANES), 1)
        row = hist[None]
        for j, s in enumerate(own_scales):
            row = row + jnp.where(l1 == scale_base + j, s, 0.0)
        hists_ref[pl.ds(my_pos, 1)] = row

        pl.semaphore_wait(barrier_sem, N_DEV - 1)

        hist_sends = []
        for dlt in range(1, N_DEV):
            tgt = lax.rem(my_pos + dlt, N_DEV)
            snd = pltpu.make_async_remote_copy(
                src_ref=hists_ref.at[pl.ds(my_pos, 1)],
                dst_ref=hists_ref.at[pl.ds(my_pos, 1)],
                send_sem=hist_send_sems.at[dlt - 1],
                recv_sem=hist_recv_sems.at[dlt - 1],
                device_id=(tgt,),
                device_id_type=pl.DeviceIdType.MESH,
            )
            snd.start()
            hist_sends.append(snd)

        w_sends = {}
        for dlt in range(1, N_DEV):
            tgt = lax.rem(my_pos + dlt, N_DEV)
            for j in range(e_per):
                sem = (dlt - 1) * e_per + j
                snd = pltpu.make_async_remote_copy(
                    src_ref=q8_ref.at[j],
                    dst_ref=comm_ref.at[dlt - 1, j],
                    send_sem=send_sems.at[sem],
                    recv_sem=recv_sems.at[sem],
                    device_id=(tgt,),
                    device_id_type=pl.DeviceIdType.MESH,
                )
                snd.start()
                w_sends[(dlt - 1, j)] = snd

        for dlt in range(1, N_DEV):
            org = lax.rem(my_pos + N_DEV - dlt, N_DEV)
            rcv = pltpu.make_async_remote_copy(
                src_ref=hists_ref.at[pl.ds(org, 1)],
                dst_ref=hists_ref.at[pl.ds(org, 1)],
                send_sem=hist_send_sems.at[dlt - 1],
                recv_sem=hist_recv_sems.at[dlt - 1],
                device_id=(org,),
                device_id_type=pl.DeviceIdType.MESH,
            )
            rcv.wait_recv()
        for snd in hist_sends:
            snd.wait_send()

        allh = hists_ref[...]
        rowq = lax.broadcasted_iota(jnp.int32, (N_DEV, LANES), 0)
        base = jnp.sum(jnp.where(rowq < my_pos, allh, 0.0), axis=0)
        rr = lax.broadcasted_iota(jnp.int32, (n_tok, n_tok), 0)
        cc = lax.broadcasted_iota(jnp.int32, (n_tok, n_tok), 1)
        tril = (cc < rr).astype(jnp.bfloat16)
        cum = jnp.dot(tril, onehot.astype(jnp.bfloat16),
                      preferred_element_type=jnp.float32)
        pos = cum + base[None, :]
        pos_tok = jnp.sum(pos * onehot, axis=1)
        kept = (pos_tok < CAP).astype(jnp.float32)
        mask = onehot * kept[:, None]

        xbf = x_ref[...].astype(jnp.bfloat16)

        def expert_out(qw, e, scale):
            mcol = jnp.sum(jnp.where(lane == e, mask, 0.0), axis=1)
            xm = xbf * (mcol * scale).astype(jnp.bfloat16)[:, None]
            return jnp.dot(xm, qw.astype(jnp.bfloat16),
                           preferred_element_type=jnp.float32)

        acc = None
        for j in range(e_per):
            p = expert_out(q8_ref[j], my_pos * e_per + j, own_scales[j])
            acc = p if acc is None else acc + p
        for s in (0, 2, 1):
            org = lax.rem(my_pos + N_DEV - (s + 1), N_DEV)
            orow = jnp.sum(jnp.where(rowq == org, allh, 0.0), axis=0)
            for j in range(e_per):
                w_sends[(s, j)].wait_recv()
                acc = acc + expert_out(comm_ref[s, j], org * e_per + j,
                                       orow[scale_base + j])
        for snd in w_sends.values():
            snd.wait_send()
        out_ref[...] = acc

    return pl.pallas_call(
        body,
        out_shape=jax.ShapeDtypeStruct((n_tok, h), jnp.float32),
        in_specs=[pl.BlockSpec(memory_space=pltpu.VMEM)] * 4,
        out_specs=pl.BlockSpec(memory_space=pltpu.VMEM),
        scratch_shapes=[
            pltpu.VMEM((N_DEV - 1, e_per, d, h), jnp.int8),
            pltpu.VMEM((e_per, d, h), jnp.int8),
            pltpu.VMEM((N_DEV, LANES), jnp.float32),
            pltpu.SemaphoreType.DMA(((N_DEV - 1) * e_per,)),
            pltpu.SemaphoreType.DMA(((N_DEV - 1) * e_per,)),
            pltpu.SemaphoreType.DMA((N_DEV - 1,)),
            pltpu.SemaphoreType.DMA((N_DEV - 1,)),
        ],
        compiler_params=pltpu.CompilerParams(collective_id=0),
    )(x, router_W, route_idx, expert_W)
